# baseline (speedup 1.0000x reference)
import functools

import jax
import jax.numpy as jnp
import numpy as np
from jax import lax
from jax.experimental import pallas as pl
from jax.experimental.pallas import tpu as pltpu
from jax.experimental.pallas import tpu_sc as plsc

N = 20000
NN = 10000
E = 320000
F = 128
C = 8
NCLS = 10
SLOPE = 0.01

NC = 2
NS = 16
NW = NC * NS
K = 80
NCH = E // (NW * K)
NPAD = 20096
SL = NPAD // NS

def _deg_body(col_hbm, zeros1_hbm, deg_out, colv, onesv, stage, acc, sem):
    cid = lax.axis_index("c")
    sid = lax.axis_index("s")
    wid = cid * NS + sid
    for i in range(K // 16):
        onesv[pl.ds(i * 16, 16)] = jnp.ones((16,), jnp.float32)
    pltpu.sync_copy(zeros1_hbm.at[pl.ds(sid * SL, SL)], stage)
    pltpu.sync_copy(stage, acc.at[pl.ds(sid * SL, SL)])
    pltpu.sync_copy(col_hbm.at[wid], colv)
    plsc.subcore_barrier()

    def fire(j, carry):
        pltpu.async_copy(onesv, acc.at[colv.at[j]], sem, add=True)
        return carry

    lax.fori_loop(0, NCH, fire, 0)

    def drain(j, carry):
        pltpu.make_async_copy(onesv, acc.at[colv.at[j]], sem).wait()
        return carry

    lax.fori_loop(0, NCH, drain, 0)
    plsc.subcore_barrier()
    pltpu.sync_copy(acc.at[pl.ds(sid * SL, SL)], stage)
    pltpu.sync_copy(stage, deg_out.at[pl.ds(cid * NPAD + sid * SL, SL)])


def _scatter_body(g_hbm, row_hbm, col_hbm, zeros8_hbm, s_out,
                  rowv, colv, rbuf, stage, acc, sem, sem2):
    cid = lax.axis_index("c")
    sid = lax.axis_index("s")
    wid = cid * NS + sid

    @pl.when(cid == 0)
    def _():
        pltpu.sync_copy(g_hbm.at[pl.ds(sid * SL, SL)], stage)
        pltpu.sync_copy(stage, acc.at[pl.ds(sid * SL, SL)])

    @pl.when(cid != 0)
    def _():
        pltpu.sync_copy(zeros8_hbm.at[pl.ds(sid * SL, SL)], stage)
        pltpu.sync_copy(stage, acc.at[pl.ds(sid * SL, SL)])

    pltpu.sync_copy(row_hbm.at[wid], rowv)
    pltpu.sync_copy(col_hbm.at[wid], colv)
    plsc.subcore_barrier()

    def fire(j, carry):
        pltpu.async_copy(g_hbm.at[rowv.at[j]], rbuf.at[j], sem)
        return carry

    lax.fori_loop(0, NCH, fire, 0)

    def relay(j, carry):
        pltpu.make_async_copy(g_hbm.at[rowv.at[j]], rbuf.at[j], sem).wait()
        pltpu.async_copy(rbuf.at[j], acc.at[colv.at[j]], sem2, add=True)
        return carry

    lax.fori_loop(0, NCH, relay, 0)

    def drain(j, carry):
        pltpu.make_async_copy(rbuf.at[j], acc.at[colv.at[j]], sem2).wait()
        return carry

    lax.fori_loop(0, NCH, drain, 0)
    plsc.subcore_barrier()
    pltpu.sync_copy(acc.at[pl.ds(sid * SL, SL)], stage)
    pltpu.sync_copy(stage, s_out.at[pl.ds(cid * NPAD + sid * SL, SL)])


def _g_body(x_ref, w_ref, d_ref, g_ref):
    hT = lax.dot_general(w_ref[...], x_ref[...], (((0,), (1,)), ((), ())),
                         preferred_element_type=jnp.float32)
    degv = d_ref[...]
    deg = degv[:NPAD] + degv[NPAD:] + 1.0
    dinv = jnp.reshape(lax.rsqrt(deg), (1, NPAD))
    g_ref[...] = jnp.pad(hT, ((0, 0), (0, NPAD - N))) * dinv


_g_call = pl.pallas_call(
    _g_body,
    out_shape=jax.ShapeDtypeStruct((C, NPAD), jnp.float32),
)


QW = NPAD * C // 128
_EXPAND = np.zeros((16, 128), np.float32)
for _k in range(16):
    _EXPAND[_k, _k * 8:(_k + 1) * 8] = 1.0
_QSEL = np.zeros((16 * NCLS, NCLS), np.float32)
for _j in range(NCLS):
    _QSEL[_j * 16:(_j + 1) * 16, _j] = 1.0
_G0 = NN * C // 128
_G1 = 2 * _G0


def _leaky(v):
    return jnp.where(v >= 0, v, SLOPE * v)


def _head_body(sp_ref, d16_ref, ex_ref, bt_ref, p_ref, b1_ref, w2l_ref,
               qs_ref, b2_ref, out_ref):
    d16 = d16_ref[0] + d16_ref[1] + 1.0
    dinv8 = jnp.dot(lax.rsqrt(d16), ex_ref[...],
                    preferred_element_type=jnp.float32)
    s = (sp_ref[0] + sp_ref[1]) * dinv8 + bt_ref[...]
    a = jnp.dot(_leaky(s), p_ref[...],
                preferred_element_type=jnp.float32) + b1_ref[0, 0]
    y = _leaky(a)
    yrep = jnp.concatenate([y] * NCLS, axis=1)
    prod = yrep * w2l_ref[...]
    rid = lax.broadcasted_iota(jnp.int32, (QW, 1), 0)
    s0 = jnp.sum(jnp.where(rid < _G0, prod, 0.0), axis=0, keepdims=True)
    s1 = jnp.sum(jnp.where((rid >= _G0) & (rid < _G1), prod, 0.0),
                 axis=0, keepdims=True)
    zrow = jnp.concatenate([s0, s1], axis=0)
    z = jnp.dot(zrow, qs_ref[...],
                preferred_element_type=jnp.float32) + b2_ref[...]
    m = jnp.max(z, axis=1, keepdims=True)
    lse = jnp.log(jnp.sum(jnp.exp(z - m), axis=1, keepdims=True)) + m
    out_ref[...] = z - lse


_head_call = pl.pallas_call(
    _head_body,
    out_shape=jax.ShapeDtypeStruct((2, NCLS), jnp.float32),
)


@functools.cache
def _sc_kernels():
    mesh = plsc.VectorSubcoreMesh(core_axis_name="c", subcore_axis_name="s",
                                  num_cores=NC, num_subcores=NS)
    deg_kernel = pl.kernel(
        _deg_body,
        out_type=jax.ShapeDtypeStruct((NC * NPAD,), jnp.float32),
        mesh=mesh,
        compiler_params=pltpu.CompilerParams(use_tc_tiling_on_sc=False),
        scratch_types=[
            pltpu.VMEM((NCH, K), jnp.int32),
            pltpu.VMEM((K,), jnp.float32),
            pltpu.VMEM((SL,), jnp.float32),
            pltpu.VMEM_SHARED((NPAD,), jnp.float32),
            pltpu.SemaphoreType.DMA,
        ],
    )
    scatter_kernel = pl.kernel(
        _scatter_body,
        out_type=jax.ShapeDtypeStruct((NC * NPAD, C), jnp.float32),
        mesh=mesh,
        compiler_params=pltpu.CompilerParams(use_tc_tiling_on_sc=False),
        scratch_types=[
            pltpu.VMEM((NCH, K), jnp.int32),
            pltpu.VMEM((NCH, K), jnp.int32),
            pltpu.VMEM((NCH, K, C), jnp.float32),
            pltpu.VMEM((SL, C), jnp.float32),
            pltpu.VMEM_SHARED((NPAD, C), jnp.float32),
            pltpu.SemaphoreType.DMA,
            pltpu.SemaphoreType.DMA,
        ],
    )
    return deg_kernel, scatter_kernel


def kernel(x, edge_index, batch, W_gcn, b_gcn, W_fc1, b_fc1, W_fc2, b_fc2):
    del batch
    rowp = edge_index[0].reshape(NW, NCH, K)
    colp = edge_index[1].reshape(NW, NCH, K)
    zeros1 = jnp.zeros((NPAD,), jnp.float32)
    zeros8 = jnp.zeros((NPAD, C), jnp.float32)

    deg_kernel, scatter_kernel = _sc_kernels()
    degp = deg_kernel(colp, zeros1)
    g = _g_call(x, W_gcn, degp).T
    sp = scatter_kernel(g, rowp, colp, zeros8)
    sp_wide = sp.reshape(NC, QW, 128)
    d16p = degp.reshape(NC, QW, 16)
    bt = jnp.tile(b_gcn, 16).reshape(1, 128)
    pmat = jnp.kron(jnp.eye(16, dtype=jnp.float32), W_fc1)
    w2r = W_fc2.reshape(_G0, 16, NCLS)
    w2l = jnp.concatenate(
        [w2r, w2r, jnp.zeros((QW - _G1, 16, NCLS), jnp.float32)],
        axis=0).transpose(0, 2, 1).reshape(QW, 16 * NCLS)
    out = _head_call(sp_wide, d16p, _EXPAND, bt, pmat,
                     b_fc1.reshape(1, 1), w2l, _QSEL, b_fc2.reshape(1, NCLS))
    return out
